# baseline (speedup 1.0000x reference)
import functools

import jax
import jax.numpy as jnp
from jax import lax
from jax.experimental import pallas as pl
from jax.experimental.pallas import tpu as pltpu
from jax.experimental.pallas import tpu_sc as plsc

G = 8


@functools.cache
def _build(B, S, V, D):
    NW = 32
    bs_w = B // NW
    n_groups = S // G
    assert bs_w * NW == B and n_groups * G == S and bs_w <= 128

    mesh = plsc.VectorSubcoreMesh(core_axis_name="c", subcore_axis_name="s")

    @functools.partial(
        pl.kernel,
        mesh=mesh,
        compiler_params=pltpu.CompilerParams(use_tc_tiling_on_sc=False),
        out_type=jax.ShapeDtypeStruct((B, S, D), jnp.float32),
        scratch_types=[
            pltpu.VMEM((S, bs_w), jnp.int32),
            pltpu.VMEM((G * bs_w, D), jnp.float32),
            pltpu.SemaphoreType.DMA,
        ],
    )
    def emb(idx_hbm, table_hbm, out_hbm, idx_v, rows_v, gsem):
        wid = lax.axis_index("s") * 2 + lax.axis_index("c")
        b0 = wid * bs_w

        pltpu.sync_copy(idx_hbm.at[:, pl.ds(b0, bs_w)], idx_v)

        def group(g, _):
            for j in range(G):
                pltpu.make_async_copy(
                    table_hbm.at[idx_v.at[g * G + j]],
                    rows_v.at[pl.ds(j * bs_w, bs_w)],
                    gsem,
                ).start()
            pltpu.make_async_copy(
                table_hbm.at[pl.ds(0, G * bs_w)], rows_v, gsem
            ).wait()
            for j in range(G):
                pltpu.sync_copy(
                    rows_v.at[pl.ds(j * bs_w, bs_w)],
                    out_hbm.at[pl.ds(b0, bs_w), g * G + j, :],
                )
            return 0

        lax.fori_loop(0, n_groups, group, 0)

    return emb


def kernel(sentence, table):
    B, S = sentence.shape
    V, D = table.shape
    out = _build(B, S, V, D)(sentence.T.astype(jnp.int32), table)
    return out.reshape(B, S * D)


# speedup vs baseline: 1.2741x; 1.2741x over previous
import functools

import jax
import jax.numpy as jnp
from jax import lax
from jax.experimental import pallas as pl
from jax.experimental.pallas import tpu as pltpu
from jax.experimental.pallas import tpu_sc as plsc

GS = 4
DP = 128


@functools.cache
def _build(B, S, V, D):
    NW = 32
    bs_w = B // NW
    n_groups = S // GS
    assert bs_w * NW == B and n_groups * GS == S

    mesh = plsc.VectorSubcoreMesh(core_axis_name="c", subcore_axis_name="s")

    @functools.partial(
        pl.kernel,
        mesh=mesh,
        compiler_params=pltpu.CompilerParams(use_tc_tiling_on_sc=False),
        out_type=jax.ShapeDtypeStruct((B, S * D), jnp.float32),
        scratch_types=[
            pltpu.VMEM((S, bs_w), jnp.int32),
            pltpu.VMEM((GS * bs_w, DP), jnp.float32),
            pltpu.SemaphoreType.DMA,
            pltpu.SemaphoreType.DMA,
        ],
    )
    def emb(idx_hbm, table_hbm, out_hbm, idx_v, pad_v, gsem, ssem):
        wid = lax.axis_index("s") * 2 + lax.axis_index("c")
        b0 = wid * bs_w

        pltpu.sync_copy(idx_hbm.at[:, pl.ds(b0, bs_w)], idx_v)

        def group(g, _):
            for j in range(GS):
                pltpu.make_async_copy(
                    table_hbm.at[idx_v.at[g * GS + j]],
                    pad_v.at[pl.ds(j * bs_w, bs_w)],
                    gsem,
                ).start()
            pltpu.make_async_copy(
                table_hbm.at[pl.ds(0, GS * bs_w)], pad_v, gsem
            ).wait()
            for j in range(GS):
                pltpu.make_async_copy(
                    pad_v.at[pl.ds(j * bs_w, bs_w), pl.ds(0, D)],
                    out_hbm.at[pl.ds(b0, bs_w),
                               pl.ds((g * GS + j) * D, D)],
                    ssem,
                ).start()
            pltpu.make_async_copy(
                pad_v.at[:, pl.ds(0, D)],
                out_hbm.at[pl.ds(0, GS * bs_w), pl.ds(0, D)],
                ssem,
            ).wait()
            return 0

        lax.fori_loop(0, n_groups, group, 0)

    return emb


def kernel(sentence, table):
    B, S = sentence.shape
    V, D = table.shape
    tpad = jnp.pad(table, ((0, 0), (0, DP - D)))
    return _build(B, S, V, D)(sentence.T.astype(jnp.int32), tpad)


# speedup vs baseline: 1.5961x; 1.2527x over previous
import functools

import jax
import jax.numpy as jnp
from jax import lax
from jax.experimental import pallas as pl
from jax.experimental.pallas import tpu as pltpu
from jax.experimental.pallas import tpu_sc as plsc

G = 8
TC_COLS = 2048


def _tc_transpose_body(in_ref, out_ref):
    x = in_ref[...]
    t = x.T.reshape(TC_COLS // 4, 4, 32)
    out_ref[...] = jnp.concatenate([t[:, 0], t[:, 1], t[:, 2], t[:, 3]],
                                   axis=1)


@functools.cache
def _build_tc(V, D):
    assert D == 32 and V * D % 128 == 0
    grid = (V + TC_COLS - 1) // TC_COLS
    return pl.pallas_call(
        _tc_transpose_body,
        grid=(grid,),
        in_specs=[pl.BlockSpec((D, TC_COLS), lambda i: (0, i))],
        out_specs=pl.BlockSpec((TC_COLS // 4, 128), lambda i: (i, 0)),
        out_shape=jax.ShapeDtypeStruct((V * D // 128, 128), jnp.float32),
    )


@functools.cache
def _build_sc(B, S, V, D):
    NW = 32
    rows_w = B // NW
    n_groups = rows_w // G
    assert rows_w * NW == B and n_groups * G == rows_w
    splits = []
    off = 0
    while off < S:
        n = min(128, S - off)
        splits.append((off, n))
        off += n
    assert all(o % 8 == 0 for o, _ in splits)

    mesh = plsc.VectorSubcoreMesh(core_axis_name="c", subcore_axis_name="s")

    @functools.partial(
        pl.kernel,
        mesh=mesh,
        compiler_params=pltpu.CompilerParams(use_tc_tiling_on_sc=False),
        out_type=jax.ShapeDtypeStruct((B * S, D), jnp.float32),
        scratch_types=[
            pltpu.VMEM((rows_w, S), jnp.int32),
            pltpu.VMEM((G * S, D), jnp.float32),
            pltpu.SemaphoreType.DMA,
        ],
    )
    def emb(idx_hbm, table_hbm, out_hbm, idx_v, rows_v, gsem):
        wid = lax.axis_index("s") * 2 + lax.axis_index("c")
        row0 = wid * rows_w

        pltpu.sync_copy(idx_hbm.at[pl.ds(row0, rows_w)], idx_v)

        def group(g, _):
            for r in range(G):
                for off, n in splits:
                    pltpu.make_async_copy(
                        table_hbm.at[idx_v.at[g * G + r, pl.ds(off, n)]],
                        rows_v.at[pl.ds(r * S + off, n)],
                        gsem,
                    ).start()
            pltpu.make_async_copy(
                table_hbm.at[pl.ds(0, G * S)], rows_v, gsem
            ).wait()
            pltpu.sync_copy(
                rows_v, out_hbm.at[pl.ds((row0 + g * G) * S, G * S)]
            )
            return 0

        lax.fori_loop(0, n_groups, group, 0)

    return emb


def kernel(sentence, table):
    B, S = sentence.shape
    V, D = table.shape
    tlin = _build_tc(V, D)(table.T).reshape(V, D)
    out = _build_sc(B, S, V, D)(sentence.astype(jnp.int32), tlin)
    return out.reshape(B, S * D)


# speedup vs baseline: 1.7411x; 1.0909x over previous
import functools

import jax
import jax.numpy as jnp
from jax import lax
from jax.experimental import pallas as pl
from jax.experimental.pallas import tpu as pltpu
from jax.experimental.pallas import tpu_sc as plsc

G = 8
CB = 512


def _tc_transpose_body(x0, x1, x2, x3, out_ref):
    for j, xr in enumerate((x0, x1, x2, x3)):
        out_ref[:, j * 32:(j + 1) * 32] = xr[...].T


@functools.cache
def _build_tc(V, D):
    assert D == 32
    n_cb = (V + CB - 1) // CB
    grid = (n_cb + 3) // 4
    Vp = grid * 4 * CB
    specs = [
        pl.BlockSpec((D, CB),
                     lambda i, j=j: (0, jnp.minimum(4 * i + j, n_cb - 1)))
        for j in range(4)
    ]
    tc = pl.pallas_call(
        _tc_transpose_body,
        grid=(grid,),
        in_specs=specs,
        out_specs=pl.BlockSpec((CB, 128), lambda i: (i, 0)),
        out_shape=jax.ShapeDtypeStruct((Vp * D // 128, 128), jnp.float32),
    )
    return tc, Vp


@functools.cache
def _build_sc(B, S, Vp, D):
    NW = 32
    rows_w = B // NW
    n_groups = rows_w // G
    assert rows_w * NW == B and n_groups * G == rows_w
    splits = []
    off = 0
    while off < S:
        n = min(128, S - off)
        splits.append((off, n))
        off += n
    assert all(o % 8 == 0 for o, _ in splits)

    mesh = plsc.VectorSubcoreMesh(core_axis_name="c", subcore_axis_name="s")

    @functools.partial(
        pl.kernel,
        mesh=mesh,
        compiler_params=pltpu.CompilerParams(use_tc_tiling_on_sc=False),
        out_type=jax.ShapeDtypeStruct((B * S, D), jnp.float32),
        scratch_types=[
            pltpu.VMEM((rows_w, S), jnp.int32),
            pltpu.VMEM((G * S, D), jnp.float32),
            pltpu.SemaphoreType.DMA,
        ],
    )
    def emb(idx_hbm, table_hbm, out_hbm, idx_v, rows_v, gsem):
        wid = lax.axis_index("s") * 2 + lax.axis_index("c")
        row0 = wid * rows_w

        pltpu.sync_copy(idx_hbm.at[pl.ds(row0, rows_w)], idx_v)

        def group(g, _):
            for r in range(G):
                for off, n in splits:
                    pltpu.make_async_copy(
                        table_hbm.at[idx_v.at[g * G + r, pl.ds(off, n)]],
                        rows_v.at[pl.ds(r * S + off, n)],
                        gsem,
                    ).start()
            pltpu.make_async_copy(
                table_hbm.at[pl.ds(0, G * S)], rows_v, gsem
            ).wait()
            pltpu.sync_copy(
                rows_v, out_hbm.at[pl.ds((row0 + g * G) * S, G * S)]
            )
            return 0

        lax.fori_loop(0, n_groups, group, 0)

    return emb


def kernel(sentence, table):
    B, S = sentence.shape
    V, D = table.shape
    tc, Vp = _build_tc(V, D)
    tt = table.T
    tlin = tc(tt, tt, tt, tt).reshape(Vp, D)
    r = sentence.astype(jnp.int32)
    c, u = r >> 9, r & 511
    v = ((c >> 2) << 11) + (u << 2) + (c & 3)
    out = _build_sc(B, S, Vp, D)(v, tlin)
    return out.reshape(B, S * D)


# speedup vs baseline: 2.3291x; 1.3377x over previous
import functools

import jax
import jax.numpy as jnp
from jax import lax
from jax.experimental import pallas as pl
from jax.experimental.pallas import tpu as pltpu
from jax.experimental.pallas import tpu_sc as plsc

G = 8
CB = 2048


def _tc_transpose_body(x0, x1, x2, x3, out_ref):
    for j, xr in enumerate((x0, x1, x2, x3)):
        out_ref[:, j * 32:(j + 1) * 32] = xr[...].T


@functools.cache
def _build_tc(V, D):
    assert D == 32
    n_cb = (V + CB - 1) // CB
    grid = (n_cb + 3) // 4
    Vp = grid * 4 * CB
    specs = [
        pl.BlockSpec((D, CB),
                     lambda i, j=j: (0, jnp.minimum(4 * i + j, n_cb - 1)))
        for j in range(4)
    ]
    tc = pl.pallas_call(
        _tc_transpose_body,
        grid=(grid,),
        in_specs=specs,
        out_specs=pl.BlockSpec((CB, 128), lambda i: (i, 0)),
        out_shape=jax.ShapeDtypeStruct((Vp * D // 128, 128), jnp.float32),
    )
    return tc, Vp


@functools.cache
def _build_sc(B, S, Vp, D):
    NW = 32
    rows_w = B // NW
    n_groups = rows_w // G
    assert rows_w * NW == B and n_groups * G == rows_w
    splits = []
    off = 0
    while off < S:
        n = min(128, S - off)
        splits.append((off, n))
        off += n
    assert all(o % 8 == 0 for o, _ in splits)

    mesh = plsc.VectorSubcoreMesh(core_axis_name="c", subcore_axis_name="s")

    @functools.partial(
        pl.kernel,
        mesh=mesh,
        compiler_params=pltpu.CompilerParams(use_tc_tiling_on_sc=False),
        out_type=jax.ShapeDtypeStruct((B * S, D), jnp.float32),
        scratch_types=[
            pltpu.VMEM((rows_w, S), jnp.int32),
            pltpu.VMEM((G * S, D), jnp.float32),
            pltpu.SemaphoreType.DMA,
        ],
    )
    def emb(idx_hbm, table_hbm, out_hbm, idx_v, rows_v, gsem):
        wid = lax.axis_index("s") * 2 + lax.axis_index("c")
        row0 = wid * rows_w

        pltpu.sync_copy(idx_hbm.at[pl.ds(row0, rows_w)], idx_v)

        def group(g, _):
            for r in range(G):
                for off, n in splits:
                    pltpu.make_async_copy(
                        table_hbm.at[idx_v.at[g * G + r, pl.ds(off, n)]],
                        rows_v.at[pl.ds(r * S + off, n)],
                        gsem,
                    ).start()
            pltpu.make_async_copy(
                table_hbm.at[pl.ds(0, G * S)], rows_v, gsem
            ).wait()
            pltpu.sync_copy(
                rows_v, out_hbm.at[pl.ds((row0 + g * G) * S, G * S)]
            )
            return 0

        lax.fori_loop(0, n_groups, group, 0)

    return emb


def kernel(sentence, table):
    B, S = sentence.shape
    V, D = table.shape
    tc, Vp = _build_tc(V, D)
    tt = table.T
    tlin = tc(tt, tt, tt, tt).reshape(Vp, D)
    sh = CB.bit_length() - 1
    r = sentence.astype(jnp.int32)
    c, u = r >> sh, r & (CB - 1)
    v = ((c >> 2) << (sh + 2)) + (u << 2) + (c & 3)
    out = _build_sc(B, S, Vp, D)(v, tlin)
    return out.reshape(B, S * D)


# speedup vs baseline: 2.3701x; 1.0176x over previous
import functools

import jax
import jax.numpy as jnp
from jax import lax
from jax.experimental import pallas as pl
from jax.experimental.pallas import tpu as pltpu
from jax.experimental.pallas import tpu_sc as plsc

G = 8
CB = 4096


def _tc_transpose_body(x0, x1, x2, x3, out_ref):
    for j, xr in enumerate((x0, x1, x2, x3)):
        out_ref[:, j * 32:(j + 1) * 32] = xr[...].T


@functools.cache
def _build_tc(V, D):
    assert D == 32
    n_cb = (V + CB - 1) // CB
    grid = (n_cb + 3) // 4
    Vp = grid * 4 * CB
    specs = [
        pl.BlockSpec((D, CB),
                     lambda i, j=j: (0, jnp.minimum(4 * i + j, n_cb - 1)))
        for j in range(4)
    ]
    tc = pl.pallas_call(
        _tc_transpose_body,
        grid=(grid,),
        in_specs=specs,
        out_specs=pl.BlockSpec((CB, 128), lambda i: (i, 0)),
        out_shape=jax.ShapeDtypeStruct((Vp * D // 128, 128), jnp.float32),
    )
    return tc, Vp


@functools.cache
def _build_sc(B, S, Vp, D):
    NW = 32
    rows_w = B // NW
    n_groups = rows_w // G
    assert rows_w * NW == B and n_groups * G == rows_w
    splits = []
    off = 0
    while off < S:
        n = min(128, S - off)
        splits.append((off, n))
        off += n
    assert all(o % 8 == 0 for o, _ in splits)

    mesh = plsc.VectorSubcoreMesh(core_axis_name="c", subcore_axis_name="s")

    @functools.partial(
        pl.kernel,
        mesh=mesh,
        compiler_params=pltpu.CompilerParams(use_tc_tiling_on_sc=False),
        out_type=jax.ShapeDtypeStruct((B * S, D), jnp.float32),
        scratch_types=[
            pltpu.VMEM((rows_w, S), jnp.int32),
            pltpu.VMEM((G * S, D), jnp.float32),
            pltpu.SemaphoreType.DMA,
        ],
    )
    def emb(idx_hbm, table_hbm, out_hbm, idx_v, rows_v, gsem):
        wid = lax.axis_index("s") * 2 + lax.axis_index("c")
        row0 = wid * rows_w

        pltpu.sync_copy(idx_hbm.at[pl.ds(row0, rows_w)], idx_v)

        def group(g, _):
            for r in range(G):
                for off, n in splits:
                    pltpu.make_async_copy(
                        table_hbm.at[idx_v.at[g * G + r, pl.ds(off, n)]],
                        rows_v.at[pl.ds(r * S + off, n)],
                        gsem,
                    ).start()
            pltpu.make_async_copy(
                table_hbm.at[pl.ds(0, G * S)], rows_v, gsem
            ).wait()
            pltpu.sync_copy(
                rows_v, out_hbm.at[pl.ds((row0 + g * G) * S, G * S)]
            )
            return 0

        lax.fori_loop(0, n_groups, group, 0)

    return emb


def kernel(sentence, table):
    B, S = sentence.shape
    V, D = table.shape
    tc, Vp = _build_tc(V, D)
    tt = table.T
    tlin = tc(tt, tt, tt, tt).reshape(Vp, D)
    sh = CB.bit_length() - 1
    r = sentence.astype(jnp.int32)
    c, u = r >> sh, r & (CB - 1)
    v = ((c >> 2) << (sh + 2)) + (u << 2) + (c & 3)
    out = _build_sc(B, S, Vp, D)(v, tlin)
    return out.reshape(B, S * D)


# speedup vs baseline: 3.4966x; 1.4753x over previous
import functools

import jax
import jax.numpy as jnp
from jax import lax
from jax.experimental import pallas as pl
from jax.experimental.pallas import tpu as pltpu
from jax.experimental.pallas import tpu_sc as plsc

G = 8
CB = 4096


def _tc_transpose_body(x0, x1, x2, x3, out_ref):
    x = jnp.concatenate([x0[...], x1[...], x2[...], x3[...]], axis=0)
    eye = jnp.eye(128, dtype=jnp.float32)
    out_ref[...] = lax.dot_general(
        x, eye, (((0,), (0,)), ((), ())),
        preferred_element_type=jnp.float32)


@functools.cache
def _build_tc(V, D):
    assert D == 32
    n_cb = (V + CB - 1) // CB
    grid = (n_cb + 3) // 4
    Vp = grid * 4 * CB
    specs = [
        pl.BlockSpec((D, CB),
                     lambda i, j=j: (0, jnp.minimum(4 * i + j, n_cb - 1)))
        for j in range(4)
    ]
    tc = pl.pallas_call(
        _tc_transpose_body,
        grid=(grid,),
        in_specs=specs,
        out_specs=pl.BlockSpec((CB, 128), lambda i: (i, 0)),
        out_shape=jax.ShapeDtypeStruct((Vp * D // 128, 128), jnp.float32),
    )
    return tc, Vp


@functools.cache
def _build_sc(B, S, Vp, D):
    NW = 32
    rows_w = B // NW
    n_groups = rows_w // G
    assert rows_w * NW == B and n_groups * G == rows_w
    splits = []
    off = 0
    while off < S:
        n = min(128, S - off)
        splits.append((off, n))
        off += n
    assert all(o % 8 == 0 for o, _ in splits)

    mesh = plsc.VectorSubcoreMesh(core_axis_name="c", subcore_axis_name="s")

    @functools.partial(
        pl.kernel,
        mesh=mesh,
        compiler_params=pltpu.CompilerParams(use_tc_tiling_on_sc=False),
        out_type=jax.ShapeDtypeStruct((B * S, D), jnp.float32),
        scratch_types=[
            pltpu.VMEM((rows_w, S), jnp.int32),
            pltpu.VMEM((G * S, D), jnp.float32),
            pltpu.SemaphoreType.DMA,
        ],
    )
    def emb(idx_hbm, table_hbm, out_hbm, idx_v, rows_v, gsem):
        wid = lax.axis_index("s") * 2 + lax.axis_index("c")
        row0 = wid * rows_w

        pltpu.sync_copy(idx_hbm.at[pl.ds(row0, rows_w)], idx_v)

        def group(g, _):
            for r in range(G):
                for off, n in splits:
                    pltpu.make_async_copy(
                        table_hbm.at[idx_v.at[g * G + r, pl.ds(off, n)]],
                        rows_v.at[pl.ds(r * S + off, n)],
                        gsem,
                    ).start()
            pltpu.make_async_copy(
                table_hbm.at[pl.ds(0, G * S)], rows_v, gsem
            ).wait()
            pltpu.sync_copy(
                rows_v, out_hbm.at[pl.ds((row0 + g * G) * S, G * S)]
            )
            return 0

        lax.fori_loop(0, n_groups, group, 0)

    return emb


def kernel(sentence, table):
    B, S = sentence.shape
    V, D = table.shape
    tc, Vp = _build_tc(V, D)
    tt = table.T
    tlin = tc(tt, tt, tt, tt).reshape(Vp, D)
    sh = CB.bit_length() - 1
    r = sentence.astype(jnp.int32)
    c, u = r >> sh, r & (CB - 1)
    v = ((c >> 2) << (sh + 2)) + (u << 2) + (c & 3)
    out = _build_sc(B, S, Vp, D)(v, tlin)
    return out.reshape(B, S * D)
